# Initial kernel scaffold; baseline (speedup 1.0000x reference)
import jax
import jax.numpy as jnp
from jax.experimental import pallas as pl


def kernel(word_x, word_edge_index, word_edge_attr, word_batch, sent_x, sent_edge_index, sent_edge_attr, sent_batch, params):
    raise NotImplementedError("write your pallas kernel here")


# speedup vs baseline: 3.1317x; 3.1317x over previous
import functools

import jax
import jax.numpy as jnp
from jax import lax
from jax.experimental import pallas as pl
from jax.experimental.pallas import tpu as pltpu
from jax.experimental.pallas import tpu_sc as plsc

D = 128
NGR = 64

_NC = 2
_NS = 16
_NWK = _NC * _NS


def _bcast_lane(v16, j):
    idx = jnp.full((16, 1), j, jnp.int32)
    dn = lax.GatherDimensionNumbers(
        offset_dims=(), collapsed_slice_dims=(0,), start_index_map=(0,))
    return lax.gather(v16, idx, dn, (1,),
                      mode=lax.GatherScatterMode.PROMISE_IN_BOUNDS)


def _scale_rows(rows_v, w_v, C):
    def blk(b, carry):
        w16 = w_v[pl.ds(b * 16, 16)]
        for j in range(16):
            wj = _bcast_lane(w16, j)
            r = b * 16 + j
            for k in range(D // 16):
                sl = pl.ds(k * 16, 16)
                rows_v[r, sl] = rows_v[r, sl] * wj
        return carry
    nfull = C // 16
    lax.fori_loop(0, nfull, blk, 0)
    if C % 16:
        w16 = w_v[pl.ds(nfull * 16, 16)]
        for j in range(C % 16):
            wj = _bcast_lane(w16, j)
            r = nfull * 16 + j
            for k in range(D // 16):
                sl = pl.ds(k * 16, 16)
                rows_v[r, sl] = rows_v[r, sl] * wj


def _make_seg_sum(npad, E, C, dual):
    epw = E // _NWK
    nch = epw // C
    rps = npad // _NS
    mesh = plsc.VectorSubcoreMesh(core_axis_name="c", subcore_axis_name="s")

    one = jax.ShapeDtypeStruct((_NC, npad, D), jnp.float32)
    out_type = (one, one) if dual else one
    c16 = ((C + 15) // 16) * 16
    scratch = [
        pltpu.VMEM((C,), jnp.int32),
        pltpu.VMEM((C,), jnp.int32),
        pltpu.VMEM((c16,), jnp.float32),
        pltpu.VMEM((C, D), jnp.float32),
        pltpu.VMEM_SHARED((npad, D), jnp.float32),
    ]
    if dual:
        scratch.append(pltpu.VMEM_SHARED((npad, D), jnp.float32))
    scratch.append(pltpu.SemaphoreType.DMA)

    @functools.partial(pl.kernel, out_type=out_type, mesh=mesh,
                       scratch_types=scratch)
    def seg(*refs):
        if dual:
            (h_hbm, src_hbm, dst_hbm, w_hbm, z_hbm, outf_hbm, outb_hbm,
             src_v, dst_v, w_v, rows_v, accf, accb, sem) = refs
        else:
            (h_hbm, src_hbm, dst_hbm, w_hbm, z_hbm, outf_hbm,
             src_v, dst_v, w_v, rows_v, accf, sem) = refs
        c = lax.axis_index("c")
        s = lax.axis_index("s")
        wid = s * _NC + c
        rsl = pl.ds(s * rps, rps)
        pltpu.sync_copy(z_hbm.at[rsl], accf.at[rsl])
        if dual:
            pltpu.sync_copy(z_hbm.at[rsl], accb.at[rsl])
        plsc.subcore_barrier()

        ebase = wid * epw

        def chunk(ci, carry):
            off = ebase + ci * C
            pltpu.sync_copy(src_hbm.at[pl.ds(off, C)], src_v)
            pltpu.sync_copy(dst_hbm.at[pl.ds(off, C)], dst_v)
            pltpu.sync_copy(w_hbm.at[pl.ds(off, C)], w_v.at[pl.ds(0, C)])
            pltpu.async_copy(h_hbm.at[src_v], rows_v, sem).wait()
            _scale_rows(rows_v, w_v, C)
            pltpu.sync_copy(rows_v, accf.at[dst_v], add=True)
            if dual:
                pltpu.async_copy(h_hbm.at[dst_v], rows_v, sem).wait()
                _scale_rows(rows_v, w_v, C)
                pltpu.sync_copy(rows_v, accb.at[src_v], add=True)
            return carry

        lax.fori_loop(0, nch, chunk, 0)
        plsc.subcore_barrier()
        pltpu.sync_copy(accf.at[rsl], outf_hbm.at[c, rsl])
        if dual:
            pltpu.sync_copy(accb.at[rsl], outb_hbm.at[c, rsl])

    return seg


def _mm(x, W, br):
    n, d = x.shape
    h = W.shape[1]

    def body(x_ref, w_ref, o_ref):
        o_ref[...] = jnp.dot(x_ref[...], w_ref[...],
                             preferred_element_type=jnp.float32)

    return pl.pallas_call(
        body,
        grid=(n // br,),
        in_specs=[pl.BlockSpec((br, d), lambda i: (i, 0)),
                  pl.BlockSpec((d, h), lambda i: (0, 0))],
        out_specs=pl.BlockSpec((br, h), lambda i: (i, 0)),
        out_shape=jax.ShapeDtypeStruct((n, h), jnp.float32),
    )(x, W)


def _combine(p, br):
    n = p.shape[1]

    def body(p_ref, o_ref):
        o_ref[...] = p_ref[0] + p_ref[1]

    return pl.pallas_call(
        body,
        grid=(n // br,),
        in_specs=[pl.BlockSpec((2, br, D), lambda i: (0, i, 0))],
        out_specs=pl.BlockSpec((br, D), lambda i: (i, 0)),
        out_shape=jax.ShapeDtypeStruct((n, D), jnp.float32),
    )(p)


def _gru_block(m, h, W):
    Wz, Wr, Wn, Uz, Ur, Un, G1, G2 = [W[i] for i in range(8)]
    f32 = jnp.float32
    z = jax.nn.sigmoid(jnp.dot(m, Wz, preferred_element_type=f32)
                       + jnp.dot(h, Uz, preferred_element_type=f32))
    r = jax.nn.sigmoid(jnp.dot(m, Wr, preferred_element_type=f32)
                       + jnp.dot(h, Ur, preferred_element_type=f32))
    n = jnp.tanh(jnp.dot(m, Wn, preferred_element_type=f32)
                 + jnp.dot(r * h, Un, preferred_element_type=f32))
    hn = (1.0 - z) * h + z * n
    a = jnp.dot(hn, G1, preferred_element_type=f32)
    return hn + (a * jax.nn.sigmoid(a)) * jnp.dot(
        hn, G2, preferred_element_type=f32)


def _gru_word(h, m1, m2p, Wst, br):
    n = h.shape[0]

    def body(h_ref, m1_ref, m2_ref, w_ref, o_ref):
        m = m1_ref[...] + 0.5 * (m2_ref[0] + m2_ref[1])
        o_ref[...] = _gru_block(m, h_ref[...], w_ref)

    return pl.pallas_call(
        body,
        grid=(n // br,),
        in_specs=[pl.BlockSpec((br, D), lambda i: (i, 0)),
                  pl.BlockSpec((br, D), lambda i: (i, 0)),
                  pl.BlockSpec((2, br, D), lambda i: (0, i, 0)),
                  pl.BlockSpec((8, D, D), lambda i: (0, 0, 0))],
        out_specs=pl.BlockSpec((br, D), lambda i: (i, 0)),
        out_shape=jax.ShapeDtypeStruct((n, D), jnp.float32),
    )(h, m1, m2p, Wst)


def _gru_sent(h, mfp, mbp, Wst, br):
    n = h.shape[0]
    f32 = jnp.float32

    def body(h_ref, mf_ref, mb_ref, w_ref, o_ref):
        m = (jnp.dot(mf_ref[0] + mf_ref[1], w_ref[0],
                     preferred_element_type=f32)
             + jnp.dot(mb_ref[0] + mb_ref[1], w_ref[1],
                       preferred_element_type=f32))
        o_ref[...] = _gru_block(m, h_ref[...], w_ref[2:])

    return pl.pallas_call(
        body,
        grid=(n // br,),
        in_specs=[pl.BlockSpec((br, D), lambda i: (i, 0)),
                  pl.BlockSpec((2, br, D), lambda i: (0, i, 0)),
                  pl.BlockSpec((2, br, D), lambda i: (0, i, 0)),
                  pl.BlockSpec((10, D, D), lambda i: (0, 0, 0))],
        out_specs=pl.BlockSpec((br, D), lambda i: (i, 0)),
        out_shape=jax.ShapeDtypeStruct((n, D), jnp.float32),
    )(h, mfp, mbp, Wst)


def _pool(h, Wout, batch3d, br):
    n = h.shape[0]
    nb = n // br
    f32 = jnp.float32

    def body(h_ref, w_ref, b_ref, sums_ref, cnt_ref):
        @pl.when(pl.program_id(0) == 0)
        def _():
            sums_ref[...] = jnp.zeros_like(sums_ref)
            cnt_ref[...] = jnp.zeros_like(cnt_ref)
        y = jnp.dot(h_ref[...], w_ref[...], preferred_element_type=f32)
        b = b_ref[0, 0, :]
        oh = (b[:, None] == lax.broadcasted_iota(jnp.int32, (1, NGR), 1)
              ).astype(f32)
        dn = (((0,), (0,)), ((), ()))
        sums_ref[...] += lax.dot_general(
            oh, y, dn, preferred_element_type=f32,
            precision=jax.lax.Precision.HIGHEST)
        cnt_ref[...] += lax.dot_general(
            oh, jnp.ones_like(y), dn, preferred_element_type=f32,
            precision=jax.lax.Precision.HIGHEST)

    return pl.pallas_call(
        body,
        grid=(nb,),
        in_specs=[pl.BlockSpec((br, D), lambda i: (i, 0)),
                  pl.BlockSpec((D, D), lambda i: (0, 0)),
                  pl.BlockSpec((1, 1, br), lambda i: (i, 0, 0))],
        out_specs=[pl.BlockSpec((NGR, D), lambda i: (0, 0)),
                   pl.BlockSpec((NGR, D), lambda i: (0, 0))],
        out_shape=[jax.ShapeDtypeStruct((NGR, D), jnp.float32),
                   jax.ShapeDtypeStruct((NGR, D), jnp.float32)],
    )(h, Wout, batch3d)


def _final(wsums, wcnt, ssums, scnt, fW2, fb, lng, lnb, cW1, cb1, cW2p,
           cb2p):
    f32 = jnp.float32

    def body(ws, wc, ss, sc_, fw, fb_, g_, b_, w1, b1, w2, b2, o_ref):
        wp = ws[...] / jnp.maximum(wc[...], 1.0)
        sp = ss[...] / jnp.maximum(sc_[...], 1.0)
        g = jax.nn.sigmoid(jnp.dot(wp, fw[0], preferred_element_type=f32)
                           + jnp.dot(sp, fw[1], preferred_element_type=f32)
                           + fb_[...])
        fused = g * wp + (1.0 - g) * sp
        mu = jnp.mean(fused, axis=-1, keepdims=True)
        xc = fused - mu
        var = jnp.mean(xc * xc, axis=-1, keepdims=True)
        hn = xc * jax.lax.rsqrt(var + 1e-5) * g_[...] + b_[...]
        hc = jax.nn.relu(jnp.dot(hn, w1[...], preferred_element_type=f32)
                         + b1[...])
        o_ref[...] = jnp.dot(hc, w2[...], preferred_element_type=f32) + b2[...]

    return pl.pallas_call(
        body,
        out_shape=jax.ShapeDtypeStruct((NGR, D), jnp.float32),
    )(wsums, wcnt, ssums, scnt, fW2, fb, lng, lnb, cW1, cb1, cW2p, cb2p)


def kernel(word_x, word_edge_index, word_edge_attr, word_batch,
           sent_x, sent_edge_index, sent_edge_attr, sent_batch, params):
    p = params
    f32 = jnp.float32

    n_word, d_in = word_x.shape
    n_sent = sent_x.shape[0]
    e_word = word_edge_index.shape[1]
    e_sent = sent_edge_index.shape[1]

    np_w = ((n_word + 255) // 256) * 256
    np_s = ((n_sent + 255) // 256) * 256

    c_w = 80
    c_s = 40

    seg_word = _make_seg_sum(np_w, e_word, c_w, dual=False)
    seg_sent = _make_seg_sum(np_s, e_sent, c_s, dual=False)

    wx = jnp.pad(word_x.astype(f32), ((0, np_w - n_word), (0, 0)))
    sx = jnp.pad(sent_x.astype(f32), ((0, np_s - n_sent), (0, 0)))
    wsrc = word_edge_index[0].astype(jnp.int32)
    wdst = word_edge_index[1].astype(jnp.int32)
    ssrc = sent_edge_index[0].astype(jnp.int32)
    sdst = sent_edge_index[1].astype(jnp.int32)
    ww = word_edge_attr.astype(f32)
    sw = sent_edge_attr.astype(f32)
    wzeros = jnp.zeros((np_w, D), f32)
    szeros = jnp.zeros((np_s, D), f32)

    pad_g = jnp.int32(NGR + 7)
    wb3 = jnp.pad(word_batch.astype(jnp.int32), (0, np_w - n_word),
                  constant_values=pad_g).reshape(np_w // 1024, 1, 1024)
    sb3 = jnp.pad(sent_batch.astype(jnp.int32), (0, np_s - n_sent),
                  constant_values=pad_g).reshape(np_s // 1024, 1, 1024)

    tok = [None]

    def chained(seg, h, src, dst, w, zeros):
        if tok[0] is not None:
            zeros = lax.optimization_barrier((zeros, tok[0]))[0]
        out = seg(h, src, dst, w, zeros)
        tok[0] = out
        return out

    hw = _mm(wx, p["w_in"], 1024)
    for l in range(3):
        pre = "w%d_" % l
        wst = jnp.stack([p[pre + nm] for nm in
                         ("Wz", "Wr", "Wn", "Uz", "Ur", "Un", "G1", "G2")])
        m1p = chained(seg_word, hw, wsrc, wdst, ww, wzeros)
        m1 = _combine(m1p, 1024)
        m2p = chained(seg_word, m1, wsrc, wdst, ww, wzeros)
        hw = _gru_word(hw, m1, m2p, wst, 1024)
    wsums, wcnt = _pool(hw, p["w_out"], wb3, 1024)

    hs = _mm(sx, p["s_in"], 1024)
    for l in range(3):
        pre = "s%d_" % l
        sst = jnp.stack([p[pre + nm] for nm in
                         ("Wf", "Wb", "Wz", "Wr", "Wn", "Uz", "Ur", "Un",
                          "G1", "G2")])
        mfp = chained(seg_sent, hs, ssrc, sdst, sw, szeros)
        mbp = chained(seg_sent, hs, sdst, ssrc, sw, szeros)
        hs = _gru_sent(hs, mfp, mbp, sst, 1024)
    ssums, scnt = _pool(hs, p["s_out"], sb3, 1024)

    n_cls = p["c_W2"].shape[1]
    fW2 = p["f_W"].reshape(2, D, D)
    cW2p = jnp.pad(p["c_W2"], ((0, 0), (0, D - n_cls)))
    cb2p = jnp.pad(p["c_b2"], (0, D - n_cls)).reshape(1, D)
    logits = _final(wsums, wcnt, ssums, scnt, fW2,
                    p["f_b"].reshape(1, D), p["ln_g"].reshape(1, D),
                    p["ln_b"].reshape(1, D), p["c_W1"],
                    p["c_b1"].reshape(1, D), cW2p, cb2p)
    return logits[:, :n_cls]
